# baseline (speedup 1.0000x reference)
import functools

import jax
import jax.numpy as jnp
import numpy as np
from jax import lax
from jax.experimental import pallas as pl
from jax.experimental.pallas import tpu as pltpu
from jax.experimental.pallas import tpu_sc as plsc

_K1 = np.int32(0)
_K2 = np.int32(42)
_K3 = np.int32(np.uint32(0) ^ np.uint32(42) ^ np.uint32(0x1BD11BDA))
_ROT = ((13, 15, 26, 6), (17, 29, 16, 24))
_TINY = np.float32(np.finfo(np.float32).tiny)
_SPAN = np.float32(np.float32(1.0) - _TINY)
_ONE_F32_BITS = np.int32(0x3F800000)

_S = 3


def _rotl(x, d):
    return lax.shift_left(x, np.int32(d)) | lax.shift_right_logical(
        x, np.int32(32 - d)
    )


def _threefry_bits(flat_i32):
    x0 = jnp.zeros_like(flat_i32) + _K1
    x1 = flat_i32 + _K2
    ks = (_K2, _K3, _K1)
    for r in range(5):
        for d in _ROT[r % 2]:
            x0 = x0 + x1
            x1 = _rotl(x1, d)
            x1 = x0 ^ x1
        x0 = x0 + ks[r % 3]
        x1 = x1 + ks[(r + 1) % 3] + np.int32(r + 1)
    return x0 ^ x1


def _gumbel_from_bits(bits):
    fb = lax.shift_right_logical(bits, np.int32(9)) | _ONE_F32_BITS
    floats = lax.bitcast_convert_type(fb, jnp.float32) - np.float32(1.0)
    u = jnp.maximum(_TINY, floats * _SPAN + _TINY)
    return -jnp.log(-jnp.log(u))


def _sample_body(rows_ref, w_ref, b_ref, out_ref, *, block_rows, n_out):
    rows = rows_ref[...]
    wt = w_ref[...]
    logits = jnp.dot(rows, wt, preferred_element_type=jnp.float32)
    logits = logits + b_ref[...]
    m = jnp.max(logits, axis=1, keepdims=True)
    e = jnp.exp(logits - m)
    probs = e / jnp.sum(e, axis=1, keepdims=True)
    lp = jnp.log(probs + np.float32(1e-20))

    r = block_rows
    width = _S * n_out
    ri = lax.broadcasted_iota(jnp.int32, (r, width), 0)
    ci = lax.broadcasted_iota(jnp.int32, (r, width), 1)
    base = pl.program_id(0) * np.int32(r)
    flat = (base + ri) * np.int32(width) + ci
    g = _gumbel_from_bits(_threefry_bits(flat))

    lp3 = jnp.concatenate([lp, lp, lp], axis=1)
    v = lp3 + g
    ki = lax.broadcasted_iota(jnp.int32, (r, n_out), 1)
    toks = []
    for s in range(_S):
        vs = v[:, s * n_out : (s + 1) * n_out]
        ms = jnp.max(vs, axis=1, keepdims=True)
        tok = jnp.min(jnp.where(vs == ms, ki, np.int32(n_out)), axis=1)
        toks.append(tok[:, None])
    out_ref[...] = jnp.concatenate(toks, axis=1)


def _sample_tc(gathered, wt, bb, *, interpret=False):
    b_rows, d = gathered.shape
    n_out = wt.shape[1]
    block_rows = 512
    grid = (b_rows // block_rows,)
    body = functools.partial(_sample_body, block_rows=block_rows, n_out=n_out)
    return pl.pallas_call(
        body,
        grid=grid,
        in_specs=[
            pl.BlockSpec((block_rows, d), lambda i: (i, 0)),
            pl.BlockSpec((d, n_out), lambda i: (0, 0)),
            pl.BlockSpec((1, n_out), lambda i: (0, 0)),
        ],
        out_specs=pl.BlockSpec((block_rows, _S), lambda i: (i, 0)),
        out_shape=jax.ShapeDtypeStruct((b_rows, _S), jnp.int32),
        compiler_params=pltpu.CompilerParams(
            dimension_semantics=("arbitrary",)
        ),
        interpret=interpret,
    )(gathered, wt, bb)


def _sc_gather(table, idx):
    n_rows, d = table.shape
    b_rows = idx.shape[0]
    info = plsc.get_sparse_core_info()
    nw = info.num_cores * info.num_subcores
    b_per_w = b_rows // nw
    mesh = plsc.VectorSubcoreMesh(core_axis_name="c", subcore_axis_name="s")

    @functools.partial(
        pl.kernel,
        mesh=mesh,
        out_type=jax.ShapeDtypeStruct((b_rows, d), jnp.float32),
        scratch_types=[
            pltpu.VMEM((b_per_w,), jnp.int32),
            pltpu.VMEM((b_per_w, d), jnp.float32),
            pltpu.SemaphoreType.DMA,
        ],
        compiler_params=pltpu.CompilerParams(use_tc_tiling_on_sc=False),
    )
    def gather_k(table_hbm, idx_hbm, out_hbm, idx_v, rows_v, sem):
        wid = lax.axis_index("s") * info.num_cores + lax.axis_index("c")
        base = wid * b_per_w
        pltpu.sync_copy(idx_hbm.at[pl.ds(base, b_per_w)], idx_v)
        pltpu.async_copy(table_hbm.at[idx_v], rows_v, sem).wait()
        pltpu.sync_copy(rows_v, out_hbm.at[pl.ds(base, b_per_w)])

    return gather_k(table, idx)


def kernel(history, embed, W, b):
    idx = history[:, -1]
    gathered = _sc_gather(embed, idx)
    wt = W.T
    bb = b.reshape(1, -1)
    return _sample_tc(gathered, wt, bb)


# speedup vs baseline: 1.3867x; 1.3867x over previous
import functools

import jax
import jax.numpy as jnp
import numpy as np
from jax import lax
from jax.experimental import pallas as pl
from jax.experimental.pallas import tpu as pltpu
from jax.experimental.pallas import tpu_sc as plsc

_K1 = np.int32(0)
_K2 = np.int32(42)
_K3 = np.int32(np.uint32(0) ^ np.uint32(42) ^ np.uint32(0x1BD11BDA))
_ROT = ((13, 15, 26, 6), (17, 29, 16, 24))
_TINY = np.float32(np.finfo(np.float32).tiny)
_SPAN = np.float32(np.float32(1.0) - _TINY)
_ONE_F32_BITS = np.int32(0x3F800000)

_S = 3
_PACK = 8


def _rotl(x, d):
    return lax.shift_left(x, np.int32(d)) | lax.shift_right_logical(
        x, np.int32(32 - d)
    )


def _threefry_bits(flat_i32):
    x0 = jnp.zeros_like(flat_i32) + _K1
    x1 = flat_i32 + _K2
    ks = (_K2, _K3, _K1)
    for r in range(5):
        for d in _ROT[r % 2]:
            x0 = x0 + x1
            x1 = _rotl(x1, d)
            x1 = x0 ^ x1
        x0 = x0 + ks[r % 3]
        x1 = x1 + ks[(r + 1) % 3] + np.int32(r + 1)
    return x0 ^ x1


def _gumbel_from_bits(bits):
    fb = lax.shift_right_logical(bits, np.int32(9)) | _ONE_F32_BITS
    floats = lax.bitcast_convert_type(fb, jnp.float32) - np.float32(1.0)
    u = jnp.maximum(_TINY, floats * _SPAN + _TINY)
    return -jnp.log(-jnp.log(u))


def _project_body(embt_ref, w_ref, b_ref, out_ref, *, cblk, n_out):
    embt = embt_ref[...]
    w = w_ref[...]
    sub = cblk // _PACK
    pieces = []
    for g in range(_PACK):
        lg = lax.dot_general(
            embt[:, g * sub : (g + 1) * sub],
            w,
            dimension_numbers=(((0,), (1,)), ((), ())),
            preferred_element_type=jnp.float32,
        )
        pieces.append(lg + b_ref[...])
    out_ref[...] = jnp.concatenate(pieces, axis=1)


def _project_tc(embt, w, bb):
    d, n_rows = embt.shape
    n_out = w.shape[0]
    cblk = 4096
    grid = (pl.cdiv(n_rows, cblk),)
    n_packed = (cblk // _PACK) * grid[0]
    body = functools.partial(_project_body, cblk=cblk, n_out=n_out)
    return pl.pallas_call(
        body,
        grid=grid,
        in_specs=[
            pl.BlockSpec((d, cblk), lambda i: (0, i)),
            pl.BlockSpec((n_out, d), lambda i: (0, 0)),
            pl.BlockSpec((1, n_out), lambda i: (0, 0)),
        ],
        out_specs=pl.BlockSpec((cblk // _PACK, _PACK * n_out), lambda i: (i, 0)),
        out_shape=jax.ShapeDtypeStruct((n_packed, _PACK * n_out), jnp.float32),
        compiler_params=pltpu.CompilerParams(
            dimension_semantics=("arbitrary",)
        ),
    )(embt, w, bb)


def _sc_gather_wide(ltab, idx):
    n_packed, lanes = ltab.shape
    b_rows = idx.shape[0]
    info = plsc.get_sparse_core_info()
    nw = info.num_cores * info.num_subcores
    b_per_w = b_rows // nw
    nl = info.num_lanes
    mesh = plsc.VectorSubcoreMesh(core_axis_name="c", subcore_axis_name="s")

    @functools.partial(
        pl.kernel,
        mesh=mesh,
        out_type=jax.ShapeDtypeStruct((b_rows, lanes), jnp.float32),
        scratch_types=[
            pltpu.VMEM((b_per_w,), jnp.int32),
            pltpu.VMEM((b_per_w,), jnp.int32),
            pltpu.VMEM((b_per_w, lanes), jnp.float32),
            pltpu.SemaphoreType.DMA,
        ],
        compiler_params=pltpu.CompilerParams(use_tc_tiling_on_sc=True),
    )
    def gather_k(ltab_hbm, idx_hbm, out_hbm, idx_v, row_v, wide_v, sem):
        wid = lax.axis_index("s") * info.num_cores + lax.axis_index("c")
        base = wid * b_per_w
        pltpu.sync_copy(idx_hbm.at[pl.ds(base, b_per_w)], idx_v)
        for j in range(b_per_w // nl):
            iv = idx_v[pl.ds(j * nl, nl)]
            row_v[pl.ds(j * nl, nl)] = lax.shift_left(
                lax.shift_right_logical(iv, 12), 9
            ) | (iv & np.int32(511))
        pltpu.async_copy(ltab_hbm.at[row_v], wide_v, sem).wait()
        pltpu.sync_copy(wide_v, out_hbm.at[pl.ds(base, b_per_w)])

    return gather_k(ltab, idx)


def _sample_body(wide_ref, idx_ref, out_ref, *, block_rows, n_out):
    wide = wide_ref[...]
    idxm = lax.shift_right_logical(idx_ref[...], 9) & np.int32(_PACK - 1)
    logits = wide[:, (_PACK - 1) * n_out : _PACK * n_out]
    for g in range(_PACK - 2, -1, -1):
        logits = jnp.where(
            idxm == np.int32(g), wide[:, g * n_out : (g + 1) * n_out], logits
        )
    m = jnp.max(logits, axis=1, keepdims=True)
    e = jnp.exp(logits - m)
    probs = e / jnp.sum(e, axis=1, keepdims=True)
    lp = jnp.log(probs + np.float32(1e-20))

    r = block_rows
    width = _S * n_out
    ri = lax.broadcasted_iota(jnp.int32, (r, width), 0)
    ci = lax.broadcasted_iota(jnp.int32, (r, width), 1)
    base = pl.program_id(0) * np.int32(r)
    flat = (base + ri) * np.int32(width) + ci
    g = _gumbel_from_bits(_threefry_bits(flat))

    lp3 = jnp.concatenate([lp, lp, lp], axis=1)
    v = lp3 + g
    ki = lax.broadcasted_iota(jnp.int32, (r, n_out), 1)
    toks = []
    for s in range(_S):
        vs = v[:, s * n_out : (s + 1) * n_out]
        ms = jnp.max(vs, axis=1, keepdims=True)
        tok = jnp.min(jnp.where(vs == ms, ki, np.int32(n_out)), axis=1)
        toks.append(tok[:, None])
    out_ref[...] = jnp.concatenate(toks, axis=1)


def _sample_tc(wide, idx2d, n_out, *, interpret=False):
    b_rows = wide.shape[0]
    block_rows = 512
    grid = (b_rows // block_rows,)
    body = functools.partial(_sample_body, block_rows=block_rows, n_out=n_out)
    return pl.pallas_call(
        body,
        grid=grid,
        in_specs=[
            pl.BlockSpec((block_rows, wide.shape[1]), lambda i: (i, 0)),
            pl.BlockSpec((block_rows, 1), lambda i: (i, 0)),
        ],
        out_specs=pl.BlockSpec((block_rows, _S), lambda i: (i, 0)),
        out_shape=jax.ShapeDtypeStruct((b_rows, _S), jnp.int32),
        compiler_params=pltpu.CompilerParams(
            dimension_semantics=("arbitrary",)
        ),
        interpret=interpret,
    )(wide, idx2d)


def kernel(history, embed, W, b):
    idx = history[:, -1]
    embt = embed.T
    ltab = _project_tc(embt, W, b.reshape(1, -1))
    wide = _sc_gather_wide(ltab, idx)
    return _sample_tc(wide, idx[:, None], W.shape[0])


# speedup vs baseline: 1.7469x; 1.2597x over previous
import functools

import jax
import jax.numpy as jnp
import numpy as np
from jax import lax
from jax.experimental import pallas as pl
from jax.experimental.pallas import tpu as pltpu
from jax.experimental.pallas import tpu_sc as plsc

_K1 = np.int32(0)
_K2 = np.int32(42)
_K3 = np.int32(np.uint32(0) ^ np.uint32(42) ^ np.uint32(0x1BD11BDA))
_ROT = ((13, 15, 26, 6), (17, 29, 16, 24))
_TINY = np.float32(np.finfo(np.float32).tiny)
_SPAN = np.float32(np.float32(1.0) - _TINY)
_ONE_F32_BITS = np.int32(0x3F800000)

_S = 3
_PACK = 8


def _rotl(x, d):
    return lax.shift_left(x, np.int32(d)) | lax.shift_right_logical(
        x, np.int32(32 - d)
    )


def _threefry_bits(flat_i32):
    x0 = jnp.zeros_like(flat_i32) + _K1
    x1 = flat_i32 + _K2
    ks = (_K2, _K3, _K1)
    for r in range(5):
        for d in _ROT[r % 2]:
            x0 = x0 + x1
            x1 = _rotl(x1, d)
            x1 = x0 ^ x1
        x0 = x0 + ks[r % 3]
        x1 = x1 + ks[(r + 1) % 3] + np.int32(r + 1)
    return x0 ^ x1


def _gumbel_from_bits(bits):
    fb = lax.shift_right_logical(bits, np.int32(9)) | _ONE_F32_BITS
    floats = lax.bitcast_convert_type(fb, jnp.float32) - np.float32(1.0)
    u = jnp.maximum(_TINY, floats * _SPAN + _TINY)
    return -jnp.log(-jnp.log(u))


def _project_body(*refs):
    (
        e0, e1, e2, e3, e4, e5, e6, e7, w8t_ref, b_ref, out_ref,
    ) = refs
    embt8 = jnp.concatenate(
        [e0[...], e1[...], e2[...], e3[...], e4[...], e5[...], e6[...], e7[...]],
        axis=0,
    )
    lg = lax.dot_general(
        embt8,
        w8t_ref[...],
        dimension_numbers=(((0,), (0,)), ((), ())),
        preferred_element_type=jnp.float32,
    )
    out_ref[...] = lg + b_ref[...]


def _project_tc(embt, w8t, b128):
    d, n_rows = embt.shape
    lanes = w8t.shape[1]
    sub = 512
    cblk = sub * _PACK
    n_i = pl.cdiv(n_rows, cblk)
    n_packed = sub * n_i
    max_blk = pl.cdiv(n_rows, sub) - 1

    def espec(g):
        return pl.BlockSpec(
            (d, sub),
            lambda i, g=g: (0, jnp.minimum(i * _PACK + g, max_blk)),
        )

    return pl.pallas_call(
        _project_body,
        grid=(n_i,),
        in_specs=[espec(g) for g in range(_PACK)]
        + [
            pl.BlockSpec((_PACK * d, lanes), lambda i: (0, 0)),
            pl.BlockSpec((1, lanes), lambda i: (0, 0)),
        ],
        out_specs=pl.BlockSpec((sub, lanes), lambda i: (i, 0)),
        out_shape=jax.ShapeDtypeStruct((n_packed, lanes), jnp.float32),
        compiler_params=pltpu.CompilerParams(
            dimension_semantics=("arbitrary",),
        ),
    )(*([embt] * _PACK), w8t, b128)


def _sc_gather_wide(ltab, idx):
    n_packed, lanes = ltab.shape
    b_rows = idx.shape[0]
    info = plsc.get_sparse_core_info()
    nw = info.num_cores * info.num_subcores
    b_per_w = b_rows // nw
    nl = info.num_lanes
    mesh = plsc.VectorSubcoreMesh(core_axis_name="c", subcore_axis_name="s")

    @functools.partial(
        pl.kernel,
        mesh=mesh,
        out_type=jax.ShapeDtypeStruct((b_rows, lanes), jnp.float32),
        scratch_types=[
            pltpu.VMEM((b_per_w,), jnp.int32),
            pltpu.VMEM((b_per_w,), jnp.int32),
            pltpu.VMEM((b_per_w, lanes), jnp.float32),
            pltpu.SemaphoreType.DMA,
        ],
        compiler_params=pltpu.CompilerParams(use_tc_tiling_on_sc=True),
    )
    def gather_k(ltab_hbm, idx_hbm, out_hbm, idx_v, row_v, wide_v, sem):
        wid = lax.axis_index("s") * info.num_cores + lax.axis_index("c")
        base = wid * b_per_w
        pltpu.sync_copy(idx_hbm.at[pl.ds(base, b_per_w)], idx_v)
        for j in range(b_per_w // nl):
            iv = idx_v[pl.ds(j * nl, nl)]
            row_v[pl.ds(j * nl, nl)] = lax.shift_left(
                lax.shift_right_logical(iv, 12), 9
            ) | (iv & np.int32(511))
        pltpu.async_copy(ltab_hbm.at[row_v], wide_v, sem).wait()
        pltpu.sync_copy(wide_v, out_hbm.at[pl.ds(base, b_per_w)])

    return gather_k(ltab, idx)


def _sample_body(wide_ref, idx_ref, out_ref, *, block_rows, n_out):
    wide = wide_ref[...]
    idxm = lax.shift_right_logical(idx_ref[...], 9) & np.int32(_PACK - 1)
    logits = wide[:, (_PACK - 1) * n_out : _PACK * n_out]
    for g in range(_PACK - 2, -1, -1):
        logits = jnp.where(
            idxm == np.int32(g), wide[:, g * n_out : (g + 1) * n_out], logits
        )
    m = jnp.max(logits, axis=1, keepdims=True)
    e = jnp.exp(logits - m)
    probs = e / jnp.sum(e, axis=1, keepdims=True)
    lp = jnp.log(probs + np.float32(1e-20))

    r = block_rows
    width = _S * n_out
    ri = lax.broadcasted_iota(jnp.int32, (r, width), 0)
    ci = lax.broadcasted_iota(jnp.int32, (r, width), 1)
    base = pl.program_id(0) * np.int32(r)
    flat = (base + ri) * np.int32(width) + ci
    g = _gumbel_from_bits(_threefry_bits(flat))

    lp3 = jnp.concatenate([lp, lp, lp], axis=1)
    v = lp3 + g
    ki = lax.broadcasted_iota(jnp.int32, (r, n_out), 1)
    toks = []
    for s in range(_S):
        vs = v[:, s * n_out : (s + 1) * n_out]
        ms = jnp.max(vs, axis=1, keepdims=True)
        tok = jnp.min(jnp.where(vs == ms, ki, np.int32(n_out)), axis=1)
        toks.append(tok[:, None])
    out_ref[...] = jnp.concatenate(toks, axis=1)


def _sample_tc(wide, idx2d, n_out, *, interpret=False):
    b_rows = wide.shape[0]
    block_rows = 512
    grid = (b_rows // block_rows,)
    body = functools.partial(_sample_body, block_rows=block_rows, n_out=n_out)
    return pl.pallas_call(
        body,
        grid=grid,
        in_specs=[
            pl.BlockSpec((block_rows, wide.shape[1]), lambda i: (i, 0)),
            pl.BlockSpec((block_rows, 1), lambda i: (i, 0)),
        ],
        out_specs=pl.BlockSpec((block_rows, _S), lambda i: (i, 0)),
        out_shape=jax.ShapeDtypeStruct((b_rows, _S), jnp.int32),
        compiler_params=pltpu.CompilerParams(
            dimension_semantics=("arbitrary",)
        ),
        interpret=interpret,
    )(wide, idx2d)


def kernel(history, embed, W, b):
    idx = history[:, -1]
    embt = embed.T
    n_out, d = W.shape
    eye8 = jnp.eye(_PACK, dtype=W.dtype)
    w8t = jnp.einsum("gh,jk->gkhj", eye8, W).reshape(_PACK * d, _PACK * n_out)
    b128 = jnp.tile(b, _PACK).reshape(1, _PACK * n_out)
    ltab = _project_tc(embt, w8t, b128)
    wide = _sc_gather_wide(ltab, idx)
    return _sample_tc(wide, idx[:, None], n_out)


# speedup vs baseline: 2.2458x; 1.2856x over previous
import functools

import jax
import jax.numpy as jnp
import numpy as np
from jax import lax
from jax.experimental import pallas as pl
from jax.experimental.pallas import tpu as pltpu
from jax.experimental.pallas import tpu_sc as plsc

_K1 = np.int32(0)
_K2 = np.int32(42)
_K3 = np.int32(np.uint32(0) ^ np.uint32(42) ^ np.uint32(0x1BD11BDA))
_ROT = ((13, 15, 26, 6), (17, 29, 16, 24))
_TINY = np.float32(np.finfo(np.float32).tiny)
_SPAN = np.float32(np.float32(1.0) - _TINY)
_ONE_F32_BITS = np.int32(0x3F800000)

_S = 3
_PACK = 8


def _rotl(x, d):
    return lax.shift_left(x, np.int32(d)) | lax.shift_right_logical(
        x, np.int32(32 - d)
    )


def _threefry_bits(flat_i32):
    x0 = jnp.zeros_like(flat_i32) + _K1
    x1 = flat_i32 + _K2
    ks = (_K2, _K3, _K1)
    for r in range(5):
        for d in _ROT[r % 2]:
            x0 = x0 + x1
            x1 = _rotl(x1, d)
            x1 = x0 ^ x1
        x0 = x0 + ks[r % 3]
        x1 = x1 + ks[(r + 1) % 3] + np.int32(r + 1)
    return x0 ^ x1


def _gumbel_from_bits(bits):
    fb = lax.shift_right_logical(bits, np.int32(9)) | _ONE_F32_BITS
    floats = lax.bitcast_convert_type(fb, jnp.float32) - np.float32(1.0)
    u = jnp.maximum(_TINY, floats * _SPAN + _TINY)
    return -jnp.log(-jnp.log(u))


def _project_body(*refs):
    (
        e0, e1, e2, e3, e4, e5, e6, e7, w8t_ref, b_ref, out_ref,
    ) = refs
    embt8 = jnp.concatenate(
        [e0[...], e1[...], e2[...], e3[...], e4[...], e5[...], e6[...], e7[...]],
        axis=0,
    )
    lg = lax.dot_general(
        embt8,
        w8t_ref[...],
        dimension_numbers=(((0,), (0,)), ((), ())),
        preferred_element_type=jnp.float32,
    )
    out_ref[...] = lg + b_ref[...]


def _project_tc(embt, w8t, b128):
    d, n_rows = embt.shape
    lanes = w8t.shape[1]
    sub = 1024
    cblk = sub * _PACK
    n_i = pl.cdiv(n_rows, cblk)
    n_packed = sub * n_i
    max_blk = pl.cdiv(n_rows, sub) - 1

    def espec(g):
        return pl.BlockSpec(
            (d, sub),
            lambda i, g=g: (0, jnp.minimum(i * _PACK + g, max_blk)),
        )

    return pl.pallas_call(
        _project_body,
        grid=(n_i,),
        in_specs=[espec(g) for g in range(_PACK)]
        + [
            pl.BlockSpec((_PACK * d, lanes), lambda i: (0, 0)),
            pl.BlockSpec((1, lanes), lambda i: (0, 0)),
        ],
        out_specs=pl.BlockSpec((sub, lanes), lambda i: (i, 0)),
        out_shape=jax.ShapeDtypeStruct((n_packed, lanes), jnp.float32),
        compiler_params=pltpu.CompilerParams(
            dimension_semantics=("arbitrary",),
        ),
    )(*([embt] * _PACK), w8t, b128)


def _sc_gather_wide(ltab, idx, sub_bits):
    n_packed, lanes = ltab.shape
    b_rows = idx.shape[0]
    info = plsc.get_sparse_core_info()
    nw = info.num_cores * info.num_subcores
    b_per_w = b_rows // nw
    nl = info.num_lanes
    mesh = plsc.VectorSubcoreMesh(core_axis_name="c", subcore_axis_name="s")

    @functools.partial(
        pl.kernel,
        mesh=mesh,
        out_type=jax.ShapeDtypeStruct((b_rows, lanes), jnp.float32),
        scratch_types=[
            pltpu.VMEM((b_per_w,), jnp.int32),
            pltpu.VMEM((b_per_w,), jnp.int32),
            pltpu.VMEM((b_per_w, lanes), jnp.float32),
            pltpu.SemaphoreType.DMA,
        ],
        compiler_params=pltpu.CompilerParams(use_tc_tiling_on_sc=True),
    )
    def gather_k(ltab_hbm, idx_hbm, out_hbm, idx_v, row_v, wide_v, sem):
        wid = lax.axis_index("s") * info.num_cores + lax.axis_index("c")
        base = wid * b_per_w
        pltpu.sync_copy(idx_hbm.at[pl.ds(base, b_per_w)], idx_v)
        for j in range(b_per_w // nl):
            iv = idx_v[pl.ds(j * nl, nl)]
            row_v[pl.ds(j * nl, nl)] = lax.shift_left(
                lax.shift_right_logical(iv, sub_bits + 3), sub_bits
            ) | (iv & np.int32((1 << sub_bits) - 1))
        pltpu.async_copy(ltab_hbm.at[row_v], wide_v, sem).wait()
        pltpu.sync_copy(wide_v, out_hbm.at[pl.ds(base, b_per_w)])

    return gather_k(ltab, idx)


def _sample_body(wide_ref, idx_ref, out_ref, *, half_rows, n_out, gshift):
    wide = wide_ref[...]
    idxm = lax.shift_right_logical(idx_ref[...], gshift) & np.int32(_PACK - 1)
    logits = wide[:, (_PACK - 1) * n_out : _PACK * n_out]
    for g in range(_PACK - 2, -1, -1):
        logits = jnp.where(
            idxm == np.int32(g), wide[:, g * n_out : (g + 1) * n_out], logits
        )
    m = jnp.max(logits, axis=1, keepdims=True)
    e = jnp.exp(logits - m)
    probs = e / jnp.sum(e, axis=1, keepdims=True)
    lp = jnp.log(probs + np.float32(1e-20))

    r = half_rows
    width = _S * n_out
    ri = lax.broadcasted_iota(jnp.int32, (r, 2 * width), 0)
    ci = lax.broadcasted_iota(jnp.int32, (r, 2 * width), 1)
    hi = ci >= np.int32(width)
    c48 = ci - jnp.where(hi, np.int32(width), np.int32(0))
    row = ri + jnp.where(hi, np.int32(r), np.int32(0))
    base = pl.program_id(0) * np.int32(2 * r)
    flat = (base + row) * np.int32(width) + c48
    g96 = _gumbel_from_bits(_threefry_bits(flat))

    lp_lo = lp[:r]
    lp_hi = lp[r:]
    lp96 = jnp.concatenate(
        [lp_lo, lp_lo, lp_lo, lp_hi, lp_hi, lp_hi], axis=1
    )
    v = lp96 + g96
    ki = lax.broadcasted_iota(jnp.int32, (r, n_out), 1)
    toks_lo, toks_hi = [], []
    for s in range(2 * _S):
        vs = v[:, s * n_out : (s + 1) * n_out]
        ms = jnp.max(vs, axis=1, keepdims=True)
        tok = jnp.min(jnp.where(vs == ms, ki, np.int32(n_out)), axis=1)
        (toks_lo if s < _S else toks_hi).append(tok[:, None])
    out_ref[...] = jnp.concatenate(
        [
            jnp.concatenate(toks_lo, axis=1),
            jnp.concatenate(toks_hi, axis=1),
        ],
        axis=0,
    )


def _sample_tc(wide, idx2d, n_out, gshift, *, interpret=False):
    b_rows = wide.shape[0]
    half_rows = 512
    block_rows = 2 * half_rows
    grid = (b_rows // block_rows,)
    body = functools.partial(
        _sample_body, half_rows=half_rows, n_out=n_out, gshift=gshift
    )
    return pl.pallas_call(
        body,
        grid=grid,
        in_specs=[
            pl.BlockSpec((block_rows, wide.shape[1]), lambda i: (i, 0)),
            pl.BlockSpec((block_rows, 1), lambda i: (i, 0)),
        ],
        out_specs=pl.BlockSpec((block_rows, _S), lambda i: (i, 0)),
        out_shape=jax.ShapeDtypeStruct((b_rows, _S), jnp.int32),
        compiler_params=pltpu.CompilerParams(
            dimension_semantics=("arbitrary",)
        ),
        interpret=interpret,
    )(wide, idx2d)


def kernel(history, embed, W, b):
    idx = history[:, -1]
    embt = embed.T
    n_out, d = W.shape
    eye8 = jnp.eye(_PACK, dtype=W.dtype)
    w8t = jnp.einsum("gh,jk->gkhj", eye8, W).reshape(_PACK * d, _PACK * n_out)
    b128 = jnp.tile(b, _PACK).reshape(1, _PACK * n_out)
    ltab = _project_tc(embt, w8t, b128)
    sub_bits = 10
    wide = _sc_gather_wide(ltab, idx, sub_bits)
    return _sample_tc(wide, idx[:, None], n_out, sub_bits)


# speedup vs baseline: 2.8416x; 1.2653x over previous
import functools

import jax
import jax.numpy as jnp
import numpy as np
from jax import lax
from jax.experimental import pallas as pl
from jax.experimental.pallas import tpu as pltpu
from jax.experimental.pallas import tpu_sc as plsc

_K1 = np.int32(0)
_K2 = np.int32(42)
_K3 = np.int32(np.uint32(0) ^ np.uint32(42) ^ np.uint32(0x1BD11BDA))
_ROT = ((13, 15, 26, 6), (17, 29, 16, 24))
_TINY = np.float32(np.finfo(np.float32).tiny)
_SPAN = np.float32(np.float32(1.0) - _TINY)
_ONE_F32_BITS = np.int32(0x3F800000)

_S = 3
_PACK = 8


def _rotl(x, d):
    return lax.shift_left(x, np.int32(d)) | lax.shift_right_logical(
        x, np.int32(32 - d)
    )


def _threefry_bits(flat_i32):
    x0 = jnp.zeros_like(flat_i32) + _K1
    x1 = flat_i32 + _K2
    ks = (_K2, _K3, _K1)
    for r in range(5):
        for d in _ROT[r % 2]:
            x0 = x0 + x1
            x1 = _rotl(x1, d)
            x1 = x0 ^ x1
        x0 = x0 + ks[r % 3]
        x1 = x1 + ks[(r + 1) % 3] + np.int32(r + 1)
    return x0 ^ x1


def _gumbel_from_bits(bits):
    fb = lax.shift_right_logical(bits, np.int32(9)) | _ONE_F32_BITS
    floats = lax.bitcast_convert_type(fb, jnp.float32) - np.float32(1.0)
    u = jnp.maximum(_TINY, floats * _SPAN + _TINY)
    return -jnp.log(-jnp.log(u))


def _project_body(*refs):
    (
        e0, e1, e2, e3, e4, e5, e6, e7, w8t_ref, b_ref, out_ref,
    ) = refs
    embt8 = jnp.concatenate(
        [e0[...], e1[...], e2[...], e3[...], e4[...], e5[...], e6[...], e7[...]],
        axis=0,
    )
    lg = lax.dot_general(
        embt8,
        w8t_ref[...],
        dimension_numbers=(((0,), (0,)), ((), ())),
        preferred_element_type=jnp.float32,
    )
    out_ref[...] = lg + b_ref[...]


def _project_tc(embt, w8t, b128):
    d, n_rows = embt.shape
    lanes = w8t.shape[1]
    sub = 1024
    cblk = sub * _PACK
    n_i = pl.cdiv(n_rows, cblk)
    n_packed = sub * n_i
    max_blk = pl.cdiv(n_rows, sub) - 1

    def espec(g):
        return pl.BlockSpec(
            (d, sub),
            lambda i, g=g: (0, jnp.minimum(i * _PACK + g, max_blk)),
        )

    return pl.pallas_call(
        _project_body,
        grid=(n_i,),
        in_specs=[espec(g) for g in range(_PACK)]
        + [
            pl.BlockSpec((_PACK * d, lanes), lambda i: (0, 0)),
            pl.BlockSpec((1, lanes), lambda i: (0, 0)),
        ],
        out_specs=pl.BlockSpec((sub, lanes), lambda i: (i, 0)),
        out_shape=jax.ShapeDtypeStruct((n_packed, lanes), jnp.float32),
        compiler_params=pltpu.CompilerParams(
            dimension_semantics=("arbitrary",),
        ),
    )(*([embt] * _PACK), w8t, b128)


def _sc_gather_wide(ltab, idx, sub_bits):
    n_packed, lanes = ltab.shape
    b_rows = idx.shape[0]
    info = plsc.get_sparse_core_info()
    nw = info.num_cores * info.num_subcores
    b_per_w = b_rows // nw
    nl = info.num_lanes
    mesh = plsc.VectorSubcoreMesh(core_axis_name="c", subcore_axis_name="s")

    @functools.partial(
        pl.kernel,
        mesh=mesh,
        out_type=jax.ShapeDtypeStruct((b_rows, lanes), jnp.float32),
        scratch_types=[
            pltpu.VMEM((b_per_w,), jnp.int32),
            pltpu.VMEM((b_per_w,), jnp.int32),
            pltpu.VMEM((b_per_w, lanes), jnp.float32),
            pltpu.SemaphoreType.DMA,
        ],
        compiler_params=pltpu.CompilerParams(use_tc_tiling_on_sc=True),
    )
    def gather_k(ltab_hbm, idx_hbm, out_hbm, idx_v, row_v, wide_v, sem):
        wid = lax.axis_index("s") * info.num_cores + lax.axis_index("c")
        base = wid * b_per_w
        pltpu.sync_copy(idx_hbm.at[pl.ds(base, b_per_w)], idx_v)
        for j in range(b_per_w // nl):
            iv = idx_v[pl.ds(j * nl, nl)]
            row_v[pl.ds(j * nl, nl)] = lax.shift_left(
                lax.shift_right_logical(iv, sub_bits + 3), sub_bits
            ) | (iv & np.int32((1 << sub_bits) - 1))
        pltpu.async_copy(ltab_hbm.at[row_v], wide_v, sem).wait()
        pltpu.sync_copy(wide_v, out_hbm.at[pl.ds(base, b_per_w)])

    return gather_k(ltab, idx)


def _sample_body(wide_ref, idx_ref, out_ref, *, half_rows, n_out, gshift):
    wide = wide_ref[...]
    idxg = lax.shift_right_logical(idx_ref[...], gshift)
    b2 = (idxg & np.int32(4)) != 0
    b1 = (idxg & np.int32(2)) != 0
    b0 = (idxg & np.int32(1)) != 0
    h = jnp.where(b2, wide[:, 64:128], wide[:, 0:64])
    q = jnp.where(b1, h[:, 32:64], h[:, 0:32])
    logits = jnp.where(b0, q[:, 16:32], q[:, 0:16])
    m = jnp.max(logits, axis=1, keepdims=True)
    e = jnp.exp(logits - m)
    probs = e / jnp.sum(e, axis=1, keepdims=True)
    lp = jnp.log(probs + np.float32(1e-20))

    r = half_rows
    width = _S * n_out
    ri = lax.broadcasted_iota(jnp.int32, (r, 2 * width), 0)
    ci = lax.broadcasted_iota(jnp.int32, (r, 2 * width), 1)
    hi = ci >= np.int32(width)
    c48 = ci - jnp.where(hi, np.int32(width), np.int32(0))
    row = ri + jnp.where(hi, np.int32(r), np.int32(0))
    base = pl.program_id(0) * np.int32(2 * r)
    flat = (base + row) * np.int32(width) + c48
    g96 = _gumbel_from_bits(_threefry_bits(flat))

    lp_lo = lp[:r]
    lp_hi = lp[r:]
    lp96 = jnp.concatenate(
        [lp_lo, lp_lo, lp_lo, lp_hi, lp_hi, lp_hi], axis=1
    )
    v = lp96 + g96
    kf = lax.broadcasted_iota(jnp.int32, (r, n_out), 1).astype(jnp.float32)
    toks_lo, toks_hi = [], []
    for s in range(2 * _S):
        vs = v[:, s * n_out : (s + 1) * n_out]
        ms = jnp.max(vs, axis=1, keepdims=True)
        tokf = jnp.min(
            jnp.where(vs == ms, kf, np.float32(n_out)), axis=1
        )
        tok = tokf.astype(jnp.int32)
        (toks_lo if s < _S else toks_hi).append(tok[:, None])
    out_ref[...] = jnp.concatenate(
        [
            jnp.concatenate(toks_lo, axis=1),
            jnp.concatenate(toks_hi, axis=1),
        ],
        axis=0,
    )


def _sample_tc(wide, idx2d, n_out, gshift, *, interpret=False):
    b_rows = wide.shape[0]
    half_rows = 1024
    block_rows = 2 * half_rows
    grid = (b_rows // block_rows,)
    body = functools.partial(
        _sample_body, half_rows=half_rows, n_out=n_out, gshift=gshift
    )
    return pl.pallas_call(
        body,
        grid=grid,
        in_specs=[
            pl.BlockSpec((block_rows, wide.shape[1]), lambda i: (i, 0)),
            pl.BlockSpec((block_rows, 1), lambda i: (i, 0)),
        ],
        out_specs=pl.BlockSpec((block_rows, _S), lambda i: (i, 0)),
        out_shape=jax.ShapeDtypeStruct((b_rows, _S), jnp.int32),
        compiler_params=pltpu.CompilerParams(
            dimension_semantics=("arbitrary",)
        ),
        interpret=interpret,
    )(wide, idx2d)


def kernel(history, embed, W, b):
    idx = history[:, -1]
    embt = embed.T
    n_out, d = W.shape
    eye8 = jnp.eye(_PACK, dtype=W.dtype)
    w8t = jnp.einsum("gh,jk->gkhj", eye8, W).reshape(_PACK * d, _PACK * n_out)
    b128 = jnp.tile(b, _PACK).reshape(1, _PACK * n_out)
    ltab = _project_tc(embt, w8t, b128)
    sub_bits = 10
    wide = _sc_gather_wide(ltab, idx, sub_bits)
    return _sample_tc(wide, idx[:, None], n_out, sub_bits)


# speedup vs baseline: 3.3672x; 1.1850x over previous
import functools

import jax
import jax.numpy as jnp
import numpy as np
from jax import lax
from jax.experimental import pallas as pl
from jax.experimental.pallas import tpu as pltpu
from jax.experimental.pallas import tpu_sc as plsc

_K1 = np.int32(0)
_K2 = np.int32(42)
_K3 = np.int32(np.uint32(0) ^ np.uint32(42) ^ np.uint32(0x1BD11BDA))
_ROT = ((13, 15, 26, 6), (17, 29, 16, 24))
_TINY = np.float32(np.finfo(np.float32).tiny)
_SPAN = np.float32(np.float32(1.0) - _TINY)
_ONE_F32_BITS = np.int32(0x3F800000)

_S = 3
_PACK = 8


def _rotl(x, d):
    return lax.shift_left(x, np.int32(d)) | lax.shift_right_logical(
        x, np.int32(32 - d)
    )


def _threefry_bits(flat_i32):
    x0 = jnp.zeros_like(flat_i32) + _K1
    x1 = flat_i32 + _K2
    ks = (_K2, _K3, _K1)
    for r in range(5):
        for d in _ROT[r % 2]:
            x0 = x0 + x1
            x1 = _rotl(x1, d)
            x1 = x0 ^ x1
        x0 = x0 + ks[r % 3]
        x1 = x1 + ks[(r + 1) % 3] + np.int32(r + 1)
    return x0 ^ x1


def _gumbel_from_bits(bits):
    fb = lax.shift_right_logical(bits, np.int32(9)) | _ONE_F32_BITS
    floats = lax.bitcast_convert_type(fb, jnp.float32) - np.float32(1.0)
    u = jnp.maximum(_TINY, floats * _SPAN + _TINY)
    return -jnp.log(-jnp.log(u))


def _project_body(*refs):
    (
        e0, e1, e2, e3, e4, e5, e6, e7, w8t_ref, b_ref, out_ref,
    ) = refs
    embt8 = jnp.concatenate(
        [e0[...], e1[...], e2[...], e3[...], e4[...], e5[...], e6[...], e7[...]],
        axis=0,
    )
    lg = lax.dot_general(
        embt8,
        w8t_ref[...],
        dimension_numbers=(((0,), (0,)), ((), ())),
        preferred_element_type=jnp.float32,
    )
    out_ref[...] = lg + b_ref[...]


def _project_tc(embt, w8t, b128):
    d, n_rows = embt.shape
    lanes = w8t.shape[1]
    sub = 2048
    cblk = sub * _PACK
    n_i = pl.cdiv(n_rows, cblk)
    n_packed = sub * n_i
    max_blk = pl.cdiv(n_rows, sub) - 1

    def espec(g):
        return pl.BlockSpec(
            (d, sub),
            lambda i, g=g: (0, jnp.minimum(i * _PACK + g, max_blk)),
        )

    return pl.pallas_call(
        _project_body,
        grid=(n_i,),
        in_specs=[espec(g) for g in range(_PACK)]
        + [
            pl.BlockSpec((_PACK * d, lanes), lambda i: (0, 0)),
            pl.BlockSpec((1, lanes), lambda i: (0, 0)),
        ],
        out_specs=pl.BlockSpec((sub, lanes), lambda i: (i, 0)),
        out_shape=jax.ShapeDtypeStruct((n_packed, lanes), jnp.float32),
        compiler_params=pltpu.CompilerParams(
            dimension_semantics=("arbitrary",),
        ),
    )(*([embt] * _PACK), w8t, b128)


def _sc_gather_wide(ltab, idx, sub_bits):
    n_packed, lanes = ltab.shape
    b_rows = idx.shape[0]
    info = plsc.get_sparse_core_info()
    nw = info.num_cores * info.num_subcores
    b_per_w = b_rows // nw
    nl = info.num_lanes
    mesh = plsc.VectorSubcoreMesh(core_axis_name="c", subcore_axis_name="s")

    @functools.partial(
        pl.kernel,
        mesh=mesh,
        out_type=jax.ShapeDtypeStruct((b_rows, lanes), jnp.float32),
        scratch_types=[
            pltpu.VMEM((b_per_w,), jnp.int32),
            pltpu.VMEM((b_per_w,), jnp.int32),
            pltpu.VMEM((b_per_w, lanes), jnp.float32),
            pltpu.SemaphoreType.DMA,
        ],
        compiler_params=pltpu.CompilerParams(use_tc_tiling_on_sc=True),
    )
    def gather_k(ltab_hbm, idx_hbm, out_hbm, idx_v, row_v, wide_v, sem):
        wid = lax.axis_index("s") * info.num_cores + lax.axis_index("c")
        base = wid * b_per_w
        pltpu.sync_copy(idx_hbm.at[pl.ds(base, b_per_w)], idx_v)
        for j in range(b_per_w // nl):
            iv = idx_v[pl.ds(j * nl, nl)]
            row_v[pl.ds(j * nl, nl)] = lax.shift_left(
                lax.shift_right_logical(iv, sub_bits + 3), sub_bits
            ) | (iv & np.int32((1 << sub_bits) - 1))
        pltpu.async_copy(ltab_hbm.at[row_v], wide_v, sem).wait()
        pltpu.sync_copy(wide_v, out_hbm.at[pl.ds(base, b_per_w)])

    return gather_k(ltab, idx)


def _sample_body(wide_ref, idx_ref, out_ref, *, half_rows, n_out, gshift):
    wide = wide_ref[...]
    idxg = lax.shift_right_logical(idx_ref[...], gshift)
    b2 = (idxg & np.int32(4)) != 0
    b1 = (idxg & np.int32(2)) != 0
    b0 = (idxg & np.int32(1)) != 0
    h = jnp.where(b2, wide[:, 64:128], wide[:, 0:64])
    q = jnp.where(b1, h[:, 32:64], h[:, 0:32])
    logits = jnp.where(b0, q[:, 16:32], q[:, 0:16])
    m = jnp.max(logits, axis=1, keepdims=True)
    e = jnp.exp(logits - m)
    probs = e / jnp.sum(e, axis=1, keepdims=True)
    lp = jnp.log(probs + np.float32(1e-20))

    r = half_rows
    width = _S * n_out
    ri = lax.broadcasted_iota(jnp.int32, (r, 2 * width), 0)
    ci = lax.broadcasted_iota(jnp.int32, (r, 2 * width), 1)
    hi = ci >= np.int32(width)
    c48 = ci - jnp.where(hi, np.int32(width), np.int32(0))
    row = ri + jnp.where(hi, np.int32(r), np.int32(0))
    base = pl.program_id(0) * np.int32(2 * r)
    flat = (base + row) * np.int32(width) + c48
    g96 = _gumbel_from_bits(_threefry_bits(flat))

    lp_lo = lp[:r]
    lp_hi = lp[r:]
    lp96 = jnp.concatenate(
        [lp_lo, lp_lo, lp_lo, lp_hi, lp_hi, lp_hi], axis=1
    )
    v = lp96 + g96
    kf = lax.broadcasted_iota(jnp.int32, (r, n_out), 1).astype(jnp.float32)
    toks_lo, toks_hi = [], []
    for s in range(2 * _S):
        vs = v[:, s * n_out : (s + 1) * n_out]
        ms = jnp.max(vs, axis=1, keepdims=True)
        tokf = jnp.min(
            jnp.where(vs == ms, kf, np.float32(n_out)), axis=1
        )
        tok = tokf.astype(jnp.int32)
        (toks_lo if s < _S else toks_hi).append(tok[:, None])
    out_ref[...] = jnp.concatenate(
        [
            jnp.concatenate(toks_lo, axis=1),
            jnp.concatenate(toks_hi, axis=1),
        ],
        axis=0,
    )


def _sample_tc(wide, idx2d, n_out, gshift, *, interpret=False):
    b_rows = wide.shape[0]
    half_rows = 1024
    block_rows = 2 * half_rows
    grid = (b_rows // block_rows,)
    body = functools.partial(
        _sample_body, half_rows=half_rows, n_out=n_out, gshift=gshift
    )
    return pl.pallas_call(
        body,
        grid=grid,
        in_specs=[
            pl.BlockSpec((block_rows, wide.shape[1]), lambda i: (i, 0)),
            pl.BlockSpec((block_rows, 1), lambda i: (i, 0)),
        ],
        out_specs=pl.BlockSpec((block_rows, _S), lambda i: (i, 0)),
        out_shape=jax.ShapeDtypeStruct((b_rows, _S), jnp.int32),
        compiler_params=pltpu.CompilerParams(
            dimension_semantics=("arbitrary",)
        ),
        interpret=interpret,
    )(wide, idx2d)


def kernel(history, embed, W, b):
    idx = history[:, -1]
    embt = embed.T
    n_out, d = W.shape
    eye8 = jnp.eye(_PACK, dtype=W.dtype)
    w8t = jnp.einsum("gh,jk->gkhj", eye8, W).reshape(_PACK * d, _PACK * n_out)
    b128 = jnp.tile(b, _PACK).reshape(1, _PACK * n_out)
    ltab = _project_tc(embt, w8t, b128)
    sub_bits = 11
    wide = _sc_gather_wide(ltab, idx, sub_bits)
    return _sample_tc(wide, idx[:, None], n_out, sub_bits)


# speedup vs baseline: 3.7492x; 1.1134x over previous
import functools

import jax
import jax.numpy as jnp
import numpy as np
from jax import lax
from jax.experimental import pallas as pl
from jax.experimental.pallas import tpu as pltpu
from jax.experimental.pallas import tpu_sc as plsc

_K1 = np.int32(0)
_K2 = np.int32(42)
_K3 = np.int32(np.uint32(0) ^ np.uint32(42) ^ np.uint32(0x1BD11BDA))
_ROT = ((13, 15, 26, 6), (17, 29, 16, 24))
_TINY = np.float32(np.finfo(np.float32).tiny)
_SPAN = np.float32(np.float32(1.0) - _TINY)
_ONE_F32_BITS = np.int32(0x3F800000)

_S = 3
_PACK = 8


def _rotl(x, d):
    return lax.shift_left(x, np.int32(d)) | lax.shift_right_logical(
        x, np.int32(32 - d)
    )


def _threefry_bits(flat_i32):
    x0 = jnp.zeros_like(flat_i32) + _K1
    x1 = flat_i32 + _K2
    ks = (_K2, _K3, _K1)
    for r in range(5):
        for d in _ROT[r % 2]:
            x0 = x0 + x1
            x1 = _rotl(x1, d)
            x1 = x0 ^ x1
        x0 = x0 + ks[r % 3]
        x1 = x1 + ks[(r + 1) % 3] + np.int32(r + 1)
    return x0 ^ x1


def _gumbel_from_bits(bits):
    fb = lax.shift_right_logical(bits, np.int32(9)) | _ONE_F32_BITS
    floats = lax.bitcast_convert_type(fb, jnp.float32) - np.float32(1.0)
    u = jnp.maximum(_TINY, floats * _SPAN + _TINY)
    return -jnp.log(-jnp.log(u))


def _project_body(*refs):
    (
        e0, e1, e2, e3, e4, e5, e6, e7, w8t_ref, b_ref, out_ref,
    ) = refs
    embt8 = jnp.concatenate(
        [e0[...], e1[...], e2[...], e3[...], e4[...], e5[...], e6[...], e7[...]],
        axis=0,
    )
    lg = lax.dot_general(
        embt8,
        w8t_ref[...],
        dimension_numbers=(((0,), (0,)), ((), ())),
        preferred_element_type=jnp.float32,
    )
    out_ref[...] = lg + b_ref[...]


def _project_tc(embt, w8t, b128):
    d, n_rows = embt.shape
    lanes = w8t.shape[1]
    sub = 4096
    cblk = sub * _PACK
    n_i = pl.cdiv(n_rows, cblk)
    n_packed = sub * n_i
    max_blk = pl.cdiv(n_rows, sub) - 1

    def espec(g):
        return pl.BlockSpec(
            (d, sub),
            lambda i, g=g: (0, jnp.minimum(i * _PACK + g, max_blk)),
        )

    return pl.pallas_call(
        _project_body,
        grid=(n_i,),
        in_specs=[espec(g) for g in range(_PACK)]
        + [
            pl.BlockSpec((_PACK * d, lanes), lambda i: (0, 0)),
            pl.BlockSpec((1, lanes), lambda i: (0, 0)),
        ],
        out_specs=pl.BlockSpec((sub, lanes), lambda i: (i, 0)),
        out_shape=jax.ShapeDtypeStruct((n_packed, lanes), jnp.float32),
        compiler_params=pltpu.CompilerParams(
            dimension_semantics=("arbitrary",),
        ),
    )(*([embt] * _PACK), w8t, b128)


def _sc_gather_wide(ltab, idx, sub_bits):
    n_packed, lanes = ltab.shape
    b_rows = idx.shape[0]
    info = plsc.get_sparse_core_info()
    nw = info.num_cores * info.num_subcores
    b_per_w = b_rows // nw
    nl = info.num_lanes
    mesh = plsc.VectorSubcoreMesh(core_axis_name="c", subcore_axis_name="s")

    @functools.partial(
        pl.kernel,
        mesh=mesh,
        out_type=jax.ShapeDtypeStruct((b_rows, lanes), jnp.float32),
        scratch_types=[
            pltpu.VMEM((b_per_w,), jnp.int32),
            pltpu.VMEM((b_per_w,), jnp.int32),
            pltpu.VMEM((b_per_w, lanes), jnp.float32),
            pltpu.SemaphoreType.DMA,
        ],
        compiler_params=pltpu.CompilerParams(use_tc_tiling_on_sc=True),
    )
    def gather_k(ltab_hbm, idx_hbm, out_hbm, idx_v, row_v, wide_v, sem):
        wid = lax.axis_index("s") * info.num_cores + lax.axis_index("c")
        base = wid * b_per_w
        pltpu.sync_copy(idx_hbm.at[pl.ds(base, b_per_w)], idx_v)
        for j in range(b_per_w // nl):
            iv = idx_v[pl.ds(j * nl, nl)]
            row_v[pl.ds(j * nl, nl)] = lax.shift_left(
                lax.shift_right_logical(iv, sub_bits + 3), sub_bits
            ) | (iv & np.int32((1 << sub_bits) - 1))
        pltpu.async_copy(ltab_hbm.at[row_v], wide_v, sem).wait()
        pltpu.sync_copy(wide_v, out_hbm.at[pl.ds(base, b_per_w)])

    return gather_k(ltab, idx)


def _sample_body(wide_ref, idx_ref, out_ref, *, half_rows, n_out, gshift):
    wide = wide_ref[...]
    idxg = lax.shift_right_logical(idx_ref[...], gshift)
    b2 = (idxg & np.int32(4)) != 0
    b1 = (idxg & np.int32(2)) != 0
    b0 = (idxg & np.int32(1)) != 0
    h = jnp.where(b2, wide[:, 64:128], wide[:, 0:64])
    q = jnp.where(b1, h[:, 32:64], h[:, 0:32])
    logits = jnp.where(b0, q[:, 16:32], q[:, 0:16])
    m = jnp.max(logits, axis=1, keepdims=True)
    e = jnp.exp(logits - m)
    probs = e / jnp.sum(e, axis=1, keepdims=True)
    lp = jnp.log(probs + np.float32(1e-20))

    r = half_rows
    width = _S * n_out
    ri = lax.broadcasted_iota(jnp.int32, (r, 2 * width), 0)
    ci = lax.broadcasted_iota(jnp.int32, (r, 2 * width), 1)
    hi = ci >= np.int32(width)
    c48 = ci - jnp.where(hi, np.int32(width), np.int32(0))
    row = ri + jnp.where(hi, np.int32(r), np.int32(0))
    base = pl.program_id(0) * np.int32(2 * r)
    flat = (base + row) * np.int32(width) + c48
    g96 = _gumbel_from_bits(_threefry_bits(flat))

    lp_lo = lp[:r]
    lp_hi = lp[r:]
    lp96 = jnp.concatenate(
        [lp_lo, lp_lo, lp_lo, lp_hi, lp_hi, lp_hi], axis=1
    )
    v = lp96 + g96
    kf = lax.broadcasted_iota(jnp.int32, (r, n_out), 1).astype(jnp.float32)
    toks_lo, toks_hi = [], []
    for s in range(2 * _S):
        vs = v[:, s * n_out : (s + 1) * n_out]
        ms = jnp.max(vs, axis=1, keepdims=True)
        tokf = jnp.min(
            jnp.where(vs == ms, kf, np.float32(n_out)), axis=1
        )
        tok = tokf.astype(jnp.int32)
        (toks_lo if s < _S else toks_hi).append(tok[:, None])
    out_ref[...] = jnp.concatenate(
        [
            jnp.concatenate(toks_lo, axis=1),
            jnp.concatenate(toks_hi, axis=1),
        ],
        axis=0,
    )


def _sample_tc(wide, idx2d, n_out, gshift, *, interpret=False):
    b_rows = wide.shape[0]
    half_rows = 1024
    block_rows = 2 * half_rows
    grid = (b_rows // block_rows,)
    body = functools.partial(
        _sample_body, half_rows=half_rows, n_out=n_out, gshift=gshift
    )
    return pl.pallas_call(
        body,
        grid=grid,
        in_specs=[
            pl.BlockSpec((block_rows, wide.shape[1]), lambda i: (i, 0)),
            pl.BlockSpec((block_rows, 1), lambda i: (i, 0)),
        ],
        out_specs=pl.BlockSpec((block_rows, _S), lambda i: (i, 0)),
        out_shape=jax.ShapeDtypeStruct((b_rows, _S), jnp.int32),
        compiler_params=pltpu.CompilerParams(
            dimension_semantics=("arbitrary",)
        ),
        interpret=interpret,
    )(wide, idx2d)


def kernel(history, embed, W, b):
    idx = history[:, -1]
    embt = embed.T
    n_out, d = W.shape
    eye8 = jnp.eye(_PACK, dtype=W.dtype)
    w8t = jnp.einsum("gh,jk->gkhj", eye8, W).reshape(_PACK * d, _PACK * n_out)
    b128 = jnp.tile(b, _PACK).reshape(1, _PACK * n_out)
    ltab = _project_tc(embt, w8t, b128)
    sub_bits = 12
    wide = _sc_gather_wide(ltab, idx, sub_bits)
    return _sample_tc(wide, idx[:, None], n_out, sub_bits)


# speedup vs baseline: 3.8364x; 1.0233x over previous
import functools

import jax
import jax.numpy as jnp
import numpy as np
from jax import lax
from jax.experimental import pallas as pl
from jax.experimental.pallas import tpu as pltpu
from jax.experimental.pallas import tpu_sc as plsc

_K1 = np.int32(0)
_K2 = np.int32(42)
_K3 = np.int32(np.uint32(0) ^ np.uint32(42) ^ np.uint32(0x1BD11BDA))
_ROT = ((13, 15, 26, 6), (17, 29, 16, 24))
_TINY = np.float32(np.finfo(np.float32).tiny)
_SPAN = np.float32(np.float32(1.0) - _TINY)
_ONE_F32_BITS = np.int32(0x3F800000)

_S = 3
_PACK = 8


def _rotl(x, d):
    return lax.shift_left(x, np.int32(d)) | lax.shift_right_logical(
        x, np.int32(32 - d)
    )


def _threefry_bits(flat_i32):
    x0 = jnp.zeros_like(flat_i32) + _K1
    x1 = flat_i32 + _K2
    ks = (_K2, _K3, _K1)
    for r in range(5):
        for d in _ROT[r % 2]:
            x0 = x0 + x1
            x1 = _rotl(x1, d)
            x1 = x0 ^ x1
        x0 = x0 + ks[r % 3]
        x1 = x1 + ks[(r + 1) % 3] + np.int32(r + 1)
    return x0 ^ x1


def _gumbel_from_bits(bits):
    fb = lax.shift_right_logical(bits, np.int32(9)) | _ONE_F32_BITS
    floats = lax.bitcast_convert_type(fb, jnp.float32) - np.float32(1.0)
    u = jnp.maximum(_TINY, floats * _SPAN + _TINY)
    return -jnp.log(-jnp.log(u))


def _project_body(*refs):
    (
        e0, e1, e2, e3, e4, e5, e6, e7, w8t_ref, b_ref, out_ref,
    ) = refs
    embt8 = jnp.concatenate(
        [e0[...], e1[...], e2[...], e3[...], e4[...], e5[...], e6[...], e7[...]],
        axis=0,
    )
    lg = lax.dot_general(
        embt8,
        w8t_ref[...],
        dimension_numbers=(((0,), (0,)), ((), ())),
        preferred_element_type=jnp.float32,
    )
    out_ref[...] = lg + b_ref[...]


def _project_tc(embt, w8t, b128):
    d, n_rows = embt.shape
    lanes = w8t.shape[1]
    sub = 8192
    cblk = sub * _PACK
    n_i = pl.cdiv(n_rows, cblk)
    n_packed = sub * n_i
    max_blk = pl.cdiv(n_rows, sub) - 1

    def espec(g):
        return pl.BlockSpec(
            (d, sub),
            lambda i, g=g: (0, jnp.minimum(i * _PACK + g, max_blk)),
        )

    return pl.pallas_call(
        _project_body,
        grid=(n_i,),
        in_specs=[espec(g) for g in range(_PACK)]
        + [
            pl.BlockSpec((_PACK * d, lanes), lambda i: (0, 0)),
            pl.BlockSpec((1, lanes), lambda i: (0, 0)),
        ],
        out_specs=pl.BlockSpec((sub, lanes), lambda i: (i, 0)),
        out_shape=jax.ShapeDtypeStruct((n_packed, lanes), jnp.float32),
        compiler_params=pltpu.CompilerParams(
            dimension_semantics=("arbitrary",),
        ),
    )(*([embt] * _PACK), w8t, b128)


def _sc_gather_wide(ltab, idx, sub_bits):
    n_packed, lanes = ltab.shape
    b_rows = idx.shape[0]
    info = plsc.get_sparse_core_info()
    nw = info.num_cores * info.num_subcores
    b_per_w = b_rows // nw
    nl = info.num_lanes
    mesh = plsc.VectorSubcoreMesh(core_axis_name="c", subcore_axis_name="s")

    @functools.partial(
        pl.kernel,
        mesh=mesh,
        out_type=jax.ShapeDtypeStruct((b_rows, lanes), jnp.float32),
        scratch_types=[
            pltpu.VMEM((b_per_w,), jnp.int32),
            pltpu.VMEM((b_per_w,), jnp.int32),
            pltpu.VMEM((b_per_w, lanes), jnp.float32),
            pltpu.SemaphoreType.DMA,
        ],
        compiler_params=pltpu.CompilerParams(use_tc_tiling_on_sc=True),
    )
    def gather_k(ltab_hbm, idx_hbm, out_hbm, idx_v, row_v, wide_v, sem):
        wid = lax.axis_index("s") * info.num_cores + lax.axis_index("c")
        base = wid * b_per_w
        pltpu.sync_copy(idx_hbm.at[pl.ds(base, b_per_w)], idx_v)
        for j in range(b_per_w // nl):
            iv = idx_v[pl.ds(j * nl, nl)]
            row_v[pl.ds(j * nl, nl)] = lax.shift_left(
                lax.shift_right_logical(iv, sub_bits + 3), sub_bits
            ) | (iv & np.int32((1 << sub_bits) - 1))
        pltpu.async_copy(ltab_hbm.at[row_v], wide_v, sem).wait()
        pltpu.sync_copy(wide_v, out_hbm.at[pl.ds(base, b_per_w)])

    return gather_k(ltab, idx)


def _sample_body(wide_ref, idx_ref, out_ref, *, half_rows, n_out, gshift):
    wide = wide_ref[...]
    idxg = lax.shift_right_logical(idx_ref[...], gshift)
    b2 = (idxg & np.int32(4)) != 0
    b1 = (idxg & np.int32(2)) != 0
    b0 = (idxg & np.int32(1)) != 0
    h = jnp.where(b2, wide[:, 64:128], wide[:, 0:64])
    q = jnp.where(b1, h[:, 32:64], h[:, 0:32])
    logits = jnp.where(b0, q[:, 16:32], q[:, 0:16])
    m = jnp.max(logits, axis=1, keepdims=True)
    e = jnp.exp(logits - m)
    probs = e / jnp.sum(e, axis=1, keepdims=True)
    lp = jnp.log(probs + np.float32(1e-20))

    r = half_rows
    width = _S * n_out
    ri = lax.broadcasted_iota(jnp.int32, (r, 2 * width), 0)
    ci = lax.broadcasted_iota(jnp.int32, (r, 2 * width), 1)
    hi = ci >= np.int32(width)
    c48 = ci - jnp.where(hi, np.int32(width), np.int32(0))
    row = ri + jnp.where(hi, np.int32(r), np.int32(0))
    base = pl.program_id(0) * np.int32(2 * r)
    flat = (base + row) * np.int32(width) + c48
    g96 = _gumbel_from_bits(_threefry_bits(flat))

    lp_lo = lp[:r]
    lp_hi = lp[r:]
    lp96 = jnp.concatenate(
        [lp_lo, lp_lo, lp_lo, lp_hi, lp_hi, lp_hi], axis=1
    )
    v = lp96 + g96
    kf = lax.broadcasted_iota(jnp.int32, (r, n_out), 1).astype(jnp.float32)
    toks_lo, toks_hi = [], []
    for s in range(2 * _S):
        vs = v[:, s * n_out : (s + 1) * n_out]
        ms = jnp.max(vs, axis=1, keepdims=True)
        tokf = jnp.min(
            jnp.where(vs == ms, kf, np.float32(n_out)), axis=1
        )
        tok = tokf.astype(jnp.int32)
        (toks_lo if s < _S else toks_hi).append(tok[:, None])
    out_ref[...] = jnp.concatenate(
        [
            jnp.concatenate(toks_lo, axis=1),
            jnp.concatenate(toks_hi, axis=1),
        ],
        axis=0,
    )


def _sample_tc(wide, idx2d, n_out, gshift, *, interpret=False):
    b_rows = wide.shape[0]
    half_rows = 1024
    block_rows = 2 * half_rows
    grid = (b_rows // block_rows,)
    body = functools.partial(
        _sample_body, half_rows=half_rows, n_out=n_out, gshift=gshift
    )
    return pl.pallas_call(
        body,
        grid=grid,
        in_specs=[
            pl.BlockSpec((block_rows, wide.shape[1]), lambda i: (i, 0)),
            pl.BlockSpec((block_rows, 1), lambda i: (i, 0)),
        ],
        out_specs=pl.BlockSpec((block_rows, _S), lambda i: (i, 0)),
        out_shape=jax.ShapeDtypeStruct((b_rows, _S), jnp.int32),
        compiler_params=pltpu.CompilerParams(
            dimension_semantics=("arbitrary",)
        ),
        interpret=interpret,
    )(wide, idx2d)


def kernel(history, embed, W, b):
    idx = history[:, -1]
    embt = embed.T
    n_out, d = W.shape
    eye8 = jnp.eye(_PACK, dtype=W.dtype)
    w8t = jnp.einsum("gh,jk->gkhj", eye8, W).reshape(_PACK * d, _PACK * n_out)
    b128 = jnp.tile(b, _PACK).reshape(1, _PACK * n_out)
    ltab = _project_tc(embt, w8t, b128)
    sub_bits = 13
    wide = _sc_gather_wide(ltab, idx, sub_bits)
    return _sample_tc(wide, idx[:, None], n_out, sub_bits)
